# baseline (speedup 1.0000x reference)
import jax
import jax.numpy as jnp
from jax.experimental import pallas as pl
from jax.experimental.pallas import tpu as pltpu

B = 8
P = 19248
C = 81
M = 32
BP = 3208
NBLK = P // BP
MROWS = B * P * M // 128
MRB = BP * M // 128
NEG_POS_RATIO = 3
BBOX_W = 1.0
MASK_W = 0.2 / 32.0


def _pass1_body(loc_ref, loct_ref, conf_ref, conft_ref, mask_hbm, maskt_hbm,
                s_ref, st_ref, mbuf, mtbuf, sem1, sem2):
    b = pl.program_id(0)
    j = pl.program_id(1)
    row0 = b * (P * M // 128) + j * MRB
    cp1 = pltpu.make_async_copy(mask_hbm.at[pl.ds(row0, MRB), :], mbuf, sem1)
    cp2 = pltpu.make_async_copy(maskt_hbm.at[pl.ds(row0, MRB), :], mtbuf,
                                sem2)
    cp1.start()
    cp2.start()

    conf = conf_ref[0]
    labels = conft_ref[0]
    amax = jnp.max(conf, axis=-1, keepdims=True)
    ex = jnp.exp(conf - amax)
    lse = jnp.log(jnp.sum(ex, axis=-1, keepdims=True)) + amax
    pos = labels > 0
    skip = pos | (labels < 0)
    s_ref[0] = jnp.where(skip, 0.0, lse - conf[:, 0:1])
    iota = jax.lax.broadcasted_iota(jnp.int32, (BP, C), 1)
    xl = jnp.sum(jnp.where(iota == labels, conf, 0.0), axis=-1, keepdims=True)
    posf = jnp.where(pos, 1.0, 0.0)
    npos = jnp.sum(posf)
    cepos = jnp.sum(jnp.where(pos, lse - xl, 0.0))

    d = jnp.abs(loc_ref[0] - loct_ref[0])
    sl1 = jnp.where(d < 1.0, 0.5 * d * d, d - 0.5)
    l_loc = jnp.sum(jnp.sum(sl1, axis=-1, keepdims=True) * posf)

    cp1.wait()
    cp2.wait()
    p = jnp.clip(mbuf[...], 1e-7, 1.0 - 1e-7)
    mt = mtbuf[...]
    a = jnp.log(p)
    bb = jnp.log1p(-p)
    bce = mt * (bb - a) - bb
    lane = jax.lax.broadcasted_iota(jnp.int32, (128, 4), 0)
    grp = jax.lax.broadcasted_iota(jnp.int32, (128, 4), 1)
    em = jnp.where(lane // M == grp, 1.0, 0.0)
    b4 = jax.lax.dot_general(bce, em, (((1,), (0,)), ((), ())),
                             preferred_element_type=jnp.float32)
    posf4 = posf.reshape(MRB, 4)
    l_mask = jnp.sum(b4 * posf4)

    ones = jnp.ones((1, 128), jnp.float32)
    st_ref[0, 0] = jnp.concatenate(
        [npos * ones, cepos * ones, l_loc * ones, l_mask * ones], axis=0)


def _select_body(s_ref, st_ref, out_ref):
    s = s_ref[...]
    np_rows = jnp.sum(st_ref[:, :, 0, 0:1], axis=1)
    np_total = jnp.sum(np_rows)
    ce_pos_tot = jnp.sum(st_ref[:, :, 1, 0:1])
    l_loc_tot = jnp.sum(st_ref[:, :, 2, 0:1])
    l_mask_tot = jnp.sum(st_ref[:, :, 3, 0:1])

    k = jnp.minimum(NEG_POS_RATIO * np_rows.astype(jnp.int32), P - 1)
    bits = jax.lax.bitcast_convert_type(s, jnp.int32)

    def t_step(i, pref):
        cand = pref | (jnp.int32(1) << (30 - i))
        cnt = jnp.sum((bits >= cand).astype(jnp.int32), axis=1, keepdims=True)
        return jnp.where(cnt >= k, cand, pref)

    t = jax.lax.fori_loop(0, 31, t_step, jnp.zeros((B, 1), jnp.int32))

    cgt = jnp.sum((bits > t).astype(jnp.int32), axis=1, keepdims=True)
    rem = k - cgt
    tie = bits == t
    idx = jax.lax.broadcasted_iota(jnp.int32, (B, P), 1)

    def j_step(i, acc):
        cand = acc | (jnp.int32(1) << (14 - i))
        cnt = jnp.sum((tie & (idx < cand)).astype(jnp.int32),
                      axis=1, keepdims=True)
        return jnp.where(cnt <= rem, cand, acc)

    j_lim = jax.lax.fori_loop(0, 15, j_step, jnp.zeros((B, 1), jnp.int32))

    sel = (bits > t) | (tie & (idx < j_lim))
    neg_sum = jnp.sum(jnp.where(sel, s, 0.0))

    n = jnp.maximum(np_total, 1.0)
    loss_l = l_loc_tot * BBOX_W / n
    loss_c = (ce_pos_tot + neg_sum) / n
    loss_m = l_mask_tot * MASK_W / n
    ones = jnp.ones((1, 128), jnp.float32)
    out_ref[...] = jnp.concatenate(
        [loss_l * ones, loss_c * ones, loss_m * ones,
         jnp.zeros((5, 128), jnp.float32)], axis=0)


def _run(loc_data, conf_data, mask_data, loc_t, conf_t, masks_t,
         interpret=False):
    conf_t3 = conf_t.reshape(B, P, 1)
    s, sta = pl.pallas_call(
        _pass1_body,
        grid=(B, NBLK),
        in_specs=[
            pl.BlockSpec((1, BP, 4), lambda b, j: (b, j, 0)),
            pl.BlockSpec((1, BP, 4), lambda b, j: (b, j, 0)),
            pl.BlockSpec((1, BP, C), lambda b, j: (b, j, 0)),
            pl.BlockSpec((1, BP, 1), lambda b, j: (b, j, 0)),
            pl.BlockSpec(memory_space=pl.ANY),
            pl.BlockSpec(memory_space=pl.ANY),
        ],
        out_specs=[
            pl.BlockSpec((1, BP, 1), lambda b, j: (b, j, 0)),
            pl.BlockSpec((1, 1, 4, 128), lambda b, j: (b, j, 0, 0)),
        ],
        out_shape=[
            jax.ShapeDtypeStruct((B, P, 1), jnp.float32),
            jax.ShapeDtypeStruct((B, NBLK, 4, 128), jnp.float32),
        ],
        scratch_shapes=[
            pltpu.VMEM((MRB, 128), jnp.float32),
            pltpu.VMEM((MRB, 128), jnp.float32),
            pltpu.SemaphoreType.DMA,
            pltpu.SemaphoreType.DMA,
        ],
        interpret=interpret,
    )(loc_data, loc_t, conf_data, conf_t3,
      mask_data.reshape(MROWS, 128), masks_t.reshape(MROWS, 128))

    out = pl.pallas_call(
        _select_body,
        out_shape=jax.ShapeDtypeStruct((8, 128), jnp.float32),
        interpret=interpret,
    )(s.reshape(B, P), sta)
    return (out[0, 0], out[1, 0], out[2, 0])


def kernel(loc_data, conf_data, mask_data, loc_t, conf_t, masks_t):
    return _run(loc_data, conf_data, mask_data, loc_t, conf_t, masks_t)


# speedup vs baseline: 1.1111x; 1.1111x over previous
import jax
import jax.numpy as jnp
from jax.experimental import pallas as pl

B = 8
P = 19248
C = 81
M = 32
BP = 6416
NBLK = P // BP
NEG_POS_RATIO = 3
BBOX_W = 1.0
MASK_W = 0.2 / 32.0


def _pass1_body(loc_ref, loct_ref, conf_ref, conft_ref, mask_ref, maskt_ref,
                s_ref, st_ref):
    conf = conf_ref[0]
    labels = conft_ref[0]
    amax = jnp.max(conf, axis=-1, keepdims=True)
    ex = jnp.exp(conf - amax)
    lse = jnp.log(jnp.sum(ex, axis=-1, keepdims=True)) + amax
    pos = labels > 0
    skip = pos | (labels < 0)
    s_ref[0] = jnp.where(skip, 0.0, lse - conf[:, 0:1])
    iota = jax.lax.broadcasted_iota(jnp.int32, (BP, C), 1)
    xl = jnp.sum(jnp.where(iota == labels, conf, 0.0), axis=-1, keepdims=True)
    posf = jnp.where(pos, 1.0, 0.0)
    npos = jnp.sum(posf)
    cepos = jnp.sum(jnp.where(pos, lse - xl, 0.0))

    d = jnp.abs(loc_ref[0] - loct_ref[0])
    sl1 = jnp.where(d < 1.0, 0.5 * d * d, d - 0.5)
    l_loc = jnp.sum(jnp.sum(sl1, axis=-1, keepdims=True) * posf)

    p = jnp.clip(mask_ref[0], 1e-7, 1.0 - 1e-7)
    mt = maskt_ref[0]
    a = jnp.log(p)
    bb = jnp.log1p(-p)
    bce = mt * (bb - a) - bb
    l_mask = jnp.sum(jnp.sum(bce, axis=-1, keepdims=True) * posf)

    ones = jnp.ones((1, 128), jnp.float32)
    st_ref[0, 0] = jnp.concatenate(
        [npos * ones, cepos * ones, l_loc * ones, l_mask * ones], axis=0)


def _select_body(s_ref, st_ref, out_ref):
    s = s_ref[...]
    np_rows = jnp.sum(st_ref[:, :, 0, 0:1], axis=1)
    np_total = jnp.sum(np_rows)
    ce_pos_tot = jnp.sum(st_ref[:, :, 1, 0:1])
    l_loc_tot = jnp.sum(st_ref[:, :, 2, 0:1])
    l_mask_tot = jnp.sum(st_ref[:, :, 3, 0:1])

    k = jnp.minimum(NEG_POS_RATIO * np_rows.astype(jnp.int32), P - 1)
    bits = jax.lax.bitcast_convert_type(s, jnp.int32)

    def t_step(i, pref):
        cand = pref | (jnp.int32(1) << (30 - i))
        cnt = jnp.sum((bits >= cand).astype(jnp.int32), axis=1, keepdims=True)
        return jnp.where(cnt >= k, cand, pref)

    t = jax.lax.fori_loop(0, 31, t_step, jnp.zeros((B, 1), jnp.int32))

    cgt = jnp.sum((bits > t).astype(jnp.int32), axis=1, keepdims=True)
    rem = k - cgt
    tie = bits == t
    idx = jax.lax.broadcasted_iota(jnp.int32, (B, P), 1)

    def j_step(i, acc):
        cand = acc | (jnp.int32(1) << (14 - i))
        cnt = jnp.sum((tie & (idx < cand)).astype(jnp.int32),
                      axis=1, keepdims=True)
        return jnp.where(cnt <= rem, cand, acc)

    j_lim = jax.lax.fori_loop(0, 15, j_step, jnp.zeros((B, 1), jnp.int32))

    sel = (bits > t) | (tie & (idx < j_lim))
    neg_sum = jnp.sum(jnp.where(sel, s, 0.0))

    n = jnp.maximum(np_total, 1.0)
    loss_l = l_loc_tot * BBOX_W / n
    loss_c = (ce_pos_tot + neg_sum) / n
    loss_m = l_mask_tot * MASK_W / n
    ones = jnp.ones((1, 128), jnp.float32)
    out_ref[...] = jnp.concatenate(
        [loss_l * ones, loss_c * ones, loss_m * ones,
         jnp.zeros((5, 128), jnp.float32)], axis=0)


def _run(loc_data, conf_data, mask_data, loc_t, conf_t, masks_t,
         interpret=False):
    conf_t3 = conf_t.reshape(B, P, 1)
    s, sta = pl.pallas_call(
        _pass1_body,
        grid=(B, NBLK),
        in_specs=[
            pl.BlockSpec((1, BP, 4), lambda b, j: (b, j, 0)),
            pl.BlockSpec((1, BP, 4), lambda b, j: (b, j, 0)),
            pl.BlockSpec((1, BP, C), lambda b, j: (b, j, 0)),
            pl.BlockSpec((1, BP, 1), lambda b, j: (b, j, 0)),
            pl.BlockSpec((1, BP, M), lambda b, j: (b, j, 0)),
            pl.BlockSpec((1, BP, M), lambda b, j: (b, j, 0)),
        ],
        out_specs=[
            pl.BlockSpec((1, BP, 1), lambda b, j: (b, j, 0)),
            pl.BlockSpec((1, 1, 4, 128), lambda b, j: (b, j, 0, 0)),
        ],
        out_shape=[
            jax.ShapeDtypeStruct((B, P, 1), jnp.float32),
            jax.ShapeDtypeStruct((B, NBLK, 4, 128), jnp.float32),
        ],
        interpret=interpret,
    )(loc_data, loc_t, conf_data, conf_t3, mask_data, masks_t)

    out = pl.pallas_call(
        _select_body,
        out_shape=jax.ShapeDtypeStruct((8, 128), jnp.float32),
        interpret=interpret,
    )(s.reshape(B, P), sta)
    return (out[0, 0], out[1, 0], out[2, 0])


def kernel(loc_data, conf_data, mask_data, loc_t, conf_t, masks_t):
    return _run(loc_data, conf_data, mask_data, loc_t, conf_t, masks_t)
